# baseline (speedup 1.0000x reference)
import functools

import jax
import jax.numpy as jnp
from jax import lax
from jax.experimental import pallas as pl
from jax.experimental.pallas import tpu as pltpu
from jax.experimental.pallas import tpu_sc as plsc

BATCH = 16384
HIDDEN = 1024
LANES = 16
QUAD = 4
NUM_CORES = 2
NUM_SUBCORES = 16
NUM_WORKERS = NUM_CORES * NUM_SUBCORES
ROWS_PER_WORKER = BATCH // NUM_WORKERS
QUADS_PER_WORKER = ROWS_PER_WORKER // QUAD
GROUPS = QUADS_PER_WORKER // LANES

_mesh = plsc.VectorSubcoreMesh(core_axis_name="c", subcore_axis_name="s")


@functools.partial(
    pl.kernel,
    mesh=_mesh,
    out_type=jax.ShapeDtypeStruct((BATCH, 1, HIDDEN), jnp.float32),
    scratch_types=[
        pltpu.VMEM((GROUPS, LANES), jnp.int32),
        pltpu.VMEM((16, QUAD, 1, HIDDEN), jnp.float32),
        pltpu.SemaphoreType.DMA,
    ],
)
def _embed_sc(pid_hbm, patt_hbm, out_hbm, pid_v, patt_v, wsem):
    wid = lax.axis_index("s") * NUM_CORES + lax.axis_index("c")
    pltpu.sync_copy(pid_hbm.at[wid], pid_v)
    pltpu.sync_copy(patt_hbm, patt_v)
    base = wid * ROWS_PER_WORKER

    handles = {}
    for g in range(GROUPS):
        pv = pid_v[g]
        for j in range(LANES):
            h = pltpu.make_async_copy(
                patt_v.at[pv[j]],
                out_hbm.at[pl.ds(base + (g * LANES + j) * QUAD, QUAD)],
                wsem)
            h.start()
            handles[g, j] = h
        if g >= 2:
            for j in range(LANES):
                handles[g - 2, j].wait()
    for g in range(GROUPS - 2, GROUPS):
        for j in range(LANES):
            handles[g, j].wait()


def kernel(x, table):
    patt = jnp.stack([
        jnp.stack([table[(p >> 3) & 1], table[(p >> 2) & 1],
                   table[(p >> 1) & 1], table[p & 1]])
        for p in range(16)
    ]).reshape(16, QUAD, 1, HIDDEN)
    q = x.reshape(-1, QUAD)
    pid = q[:, 0] * 8 + q[:, 1] * 4 + q[:, 2] * 2 + q[:, 3]
    return _embed_sc(pid.reshape(NUM_WORKERS, GROUPS, LANES), patt)


# speedup vs baseline: 1.1347x; 1.1347x over previous
import functools

import jax
import jax.numpy as jnp
from jax import lax
from jax.experimental import pallas as pl
from jax.experimental.pallas import tpu as pltpu
from jax.experimental.pallas import tpu_sc as plsc

BATCH = 16384
HIDDEN = 1024
LANES = 16
PAIR = 2
NUM_CORES = 2
NUM_SUBCORES = 16
NUM_WORKERS = NUM_CORES * NUM_SUBCORES
ROWS_PER_WORKER = BATCH // NUM_WORKERS
PAIRS_PER_WORKER = ROWS_PER_WORKER // PAIR
GROUPS = PAIRS_PER_WORKER // LANES

_mesh = plsc.VectorSubcoreMesh(core_axis_name="c", subcore_axis_name="s")


@functools.partial(
    pl.kernel,
    mesh=_mesh,
    out_type=jax.ShapeDtypeStruct((BATCH, 1, HIDDEN), jnp.float32),
    scratch_types=[
        pltpu.VMEM((GROUPS, LANES), jnp.int32),
        pltpu.VMEM((4, PAIR, 1, HIDDEN), jnp.float32),
        pltpu.SemaphoreType.DMA,
    ],
)
def _embed_sc(pid_hbm, patt_hbm, out_hbm, pid_v, patt_v, wsem):
    wid = lax.axis_index("s") * NUM_CORES + lax.axis_index("c")
    pltpu.sync_copy(pid_hbm.at[wid], pid_v)
    pltpu.sync_copy(patt_hbm, patt_v)
    base = wid * ROWS_PER_WORKER

    handles = {}
    for g in range(GROUPS):
        pv = pid_v[g]
        for j in range(LANES):
            h = pltpu.make_async_copy(
                patt_v.at[pv[j]],
                out_hbm.at[pl.ds(base + (g * LANES + j) * PAIR, PAIR)],
                wsem)
            h.start()
            handles[g, j] = h
        if g >= 2:
            for j in range(LANES):
                handles[g - 2, j].wait()
    for g in range(GROUPS - 2, GROUPS):
        for j in range(LANES):
            handles[g, j].wait()


def kernel(x, table):
    patt = jnp.stack([
        jnp.stack([table[(p >> 1) & 1], table[p & 1]]) for p in range(4)
    ]).reshape(4, PAIR, 1, HIDDEN)
    q = x.reshape(-1, PAIR)
    pid = q[:, 0] * 2 + q[:, 1]
    return _embed_sc(pid.reshape(NUM_WORKERS, GROUPS, LANES), patt)


# speedup vs baseline: 1.1626x; 1.0246x over previous
import functools

import jax
import jax.numpy as jnp
from jax import lax
from jax.experimental import pallas as pl
from jax.experimental.pallas import tpu as pltpu
from jax.experimental.pallas import tpu_sc as plsc

BATCH = 16384
HIDDEN = 1024
LANES = 16
NUM_CORES = 2
NUM_SUBCORES = 16
NUM_WORKERS = NUM_CORES * NUM_SUBCORES
ROWS_PER_WORKER = BATCH // NUM_WORKERS
GROUPS = ROWS_PER_WORKER // LANES

_mesh = plsc.VectorSubcoreMesh(core_axis_name="c", subcore_axis_name="s")


@functools.partial(
    pl.kernel,
    mesh=_mesh,
    out_type=jax.ShapeDtypeStruct((BATCH, 1, HIDDEN), jnp.float32),
    scratch_types=[
        pltpu.VMEM((GROUPS, LANES), jnp.int32),
        pltpu.VMEM((2, 1, HIDDEN), jnp.float32),
        pltpu.SemaphoreType.DMA,
    ],
)
def _embed_sc(x_hbm, table_hbm, out_hbm, idx_v, table_v, wsem):
    wid = lax.axis_index("s") * NUM_CORES + lax.axis_index("c")
    pltpu.sync_copy(x_hbm.at[wid], idx_v)
    pltpu.sync_copy(table_hbm, table_v)
    base = wid * ROWS_PER_WORKER

    handles = {}
    for g in range(GROUPS):
        xv = idx_v[g]
        for j in range(LANES):
            h = pltpu.make_async_copy(
                table_v.at[xv[j]], out_hbm.at[base + g * LANES + j], wsem)
            h.start()
            handles[g, j] = h
        if g >= 3:
            for j in range(LANES):
                handles[g - 3, j].wait()
    for g in range(GROUPS - 3, GROUPS):
        for j in range(LANES):
            handles[g, j].wait()


def kernel(x, table):
    xr = x.reshape(NUM_WORKERS, GROUPS, LANES)
    return _embed_sc(xr, table.reshape(2, 1, HIDDEN))


# speedup vs baseline: 1.1631x; 1.0004x over previous
import functools

import jax
import jax.numpy as jnp
from jax import lax
from jax.experimental import pallas as pl
from jax.experimental.pallas import tpu as pltpu
from jax.experimental.pallas import tpu_sc as plsc

BATCH = 16384
HIDDEN = 1024
LANES = 16
NUM_CORES = 2
NUM_SUBCORES = 16
NUM_WORKERS = NUM_CORES * NUM_SUBCORES
ROWS_PER_WORKER = BATCH // NUM_WORKERS
GROUPS = ROWS_PER_WORKER // LANES

_mesh = plsc.VectorSubcoreMesh(core_axis_name="c", subcore_axis_name="s")


@functools.partial(
    pl.kernel,
    mesh=_mesh,
    out_type=jax.ShapeDtypeStruct((BATCH, 1, HIDDEN), jnp.float32),
    scratch_types=[
        pltpu.VMEM((GROUPS, LANES), jnp.int32),
        pltpu.VMEM((2, 1, HIDDEN), jnp.float32),
        pltpu.SemaphoreType.DMA,
    ],
)
def _embed_sc(x_hbm, table_hbm, out_hbm, idx_v, table_v, wsem):
    wid = lax.axis_index("s") * NUM_CORES + lax.axis_index("c")
    pltpu.sync_copy(x_hbm.at[wid], idx_v)
    pltpu.sync_copy(table_hbm, table_v)
    base = wid * ROWS_PER_WORKER

    handles = {}
    for g in range(GROUPS):
        xv = idx_v[g]
        for j in range(LANES):
            h = pltpu.make_async_copy(
                table_v.at[xv[j]], out_hbm.at[base + g * LANES + j], wsem)
            h.start()
            handles[g, j] = h
        if g >= 2:
            for j in range(LANES):
                handles[g - 2, j].wait()
    for g in range(GROUPS - 2, GROUPS):
        for j in range(LANES):
            handles[g, j].wait()


def kernel(x, table):
    xr = x.reshape(NUM_WORKERS, GROUPS, LANES)
    return _embed_sc(xr, table.reshape(2, 1, HIDDEN))
